# baseline (speedup 1.0000x reference)
import functools

import jax
import jax.numpy as jnp
from jax import lax
from jax.experimental import pallas as pl
from jax.experimental.pallas import tpu as pltpu
from jax.experimental.pallas import tpu_sc as plsc

N = 50000
D = 128
R = 6
NP = 50688
CH = 8448
SH = CH
ZROW = 50000
E_TOT = 300000
EPT = 18944
ETP = 16 * EPT
SEG = 1184
SEGS = EPT // SEG
NVS = SEG // 16
FB = 48
BR = 1584
PAD_U = 1 << 20

_mesh = plsc.VectorSubcoreMesh(
    core_axis_name="c", subcore_axis_name="s", num_cores=2, num_subcores=16
)


@functools.partial(
    pl.kernel,
    out_type=jax.ShapeDtypeStruct((NP, D), jnp.float32),
    mesh=_mesh,
    compiler_params=pltpu.CompilerParams(needs_layout_passes=False),
    scratch_types=[
        pltpu.VMEM((SEG,), jnp.int32),
        pltpu.VMEM((SEG,), jnp.int32),
        pltpu.VMEM((128,), jnp.int32),
        pltpu.VMEM((128,), jnp.int32),
        pltpu.VMEM((8, FB), jnp.int32),
        pltpu.VMEM((8, FB), jnp.int32),
        pltpu.VMEM((8 * FB, D), jnp.float32),
        pltpu.VMEM_SHARED((SH, D), jnp.float32),
        pltpu.SemaphoreType.DMA,
        pltpu.SemaphoreType.DMA,
    ],
)
def _sc_scatter(xcat, u_all, g_all, s_out,
                u_seg, g_seg, vbuf, gbuf, vidx, gidx, rows_v, shared,
                sem, sem2):
    c = lax.axis_index("c")
    s = lax.axis_index("s")
    ones16 = jnp.ones((16,), jnp.int32)
    zeros16i = jnp.zeros((16,), jnp.int32)
    zrow16 = jnp.full((16,), ZROW, jnp.int32)

    ebase = s * EPT
    zb = s * 528
    ob_local = s * 528

    def _gather_desc(slot):
        return pltpu.make_async_copy(
            xcat.at[gidx.at[slot]], rows_v.at[pl.ds(slot * FB, FB)], sem)

    def _scatter_desc(slot):
        return pltpu.make_async_copy(
            rows_v.at[pl.ds(slot * FB, FB)], shared.at[vidx.at[slot]], sem2)

    for lc in range(3):
        chunk = 3 * c + lc
        lo = chunk * CH

        zeros16f = jnp.zeros((16,), jnp.float32)
        for i in range(16):
            for j in range(8):
                rows_v[i, pl.ds(j * 16, 16)] = zeros16f

        def _zero(k, _):
            pltpu.sync_copy(rows_v.at[pl.ds(0, 16)],
                            shared.at[pl.ds(zb + k * 16, 16)])
            return 0
        lax.fori_loop(0, 33, _zero, 0)
        plsc.subcore_barrier()

        def _seg(si, carry):
            pltpu.sync_copy(u_all.at[pl.ds(ebase + si * SEG, SEG)], u_seg)
            pltpu.sync_copy(g_all.at[pl.ds(ebase + si * SEG, SEG)], g_seg)

            def _vreg(i, carry):
                cnt, fc = carry
                u16 = u_seg[pl.ds(i * 16, 16)]
                g16 = g_seg[pl.ds(i * 16, 16)]
                m = (u16 >= lo) & (u16 < lo + CH)
                m32 = jnp.where(m, ones16, zeros16i)
                pos = cnt + plsc.cumsum(m32) - 1
                plsc.store_scatter(vbuf, [pos], u16 - lo, mask=m)
                plsc.store_scatter(gbuf, [pos], g16, mask=m)
                cnt2 = cnt + jnp.sum(m32)

                @pl.when(cnt2 >= FB)
                def _():
                    slot = fc & 7
                    prev7 = (fc + 1) & 7

                    @pl.when(fc >= 8)
                    def _():
                        _scatter_desc(slot).wait()
                    for tt in range(FB // 16):
                        vidx[slot, pl.ds(tt * 16, 16)] = \
                            vbuf[pl.ds(tt * 16, 16)]
                        gidx[slot, pl.ds(tt * 16, 16)] = \
                            gbuf[pl.ds(tt * 16, 16)]
                    vbuf[pl.ds(0, 16)] = vbuf[pl.ds(FB, 16)]
                    gbuf[pl.ds(0, 16)] = gbuf[pl.ds(FB, 16)]

                    @pl.when(fc >= 7)
                    def _():
                        _gather_desc(prev7).wait()
                        pltpu.async_copy(
                            rows_v.at[pl.ds(prev7 * FB, FB)],
                            shared.at[vidx.at[prev7]], sem2, add=True)
                    pltpu.async_copy(
                        xcat.at[gidx.at[slot]],
                        rows_v.at[pl.ds(slot * FB, FB)], sem)
                hit = cnt2 >= FB
                return (jnp.where(hit, cnt2 - FB, cnt2),
                        jnp.where(hit, fc + 1, fc))
            return lax.fori_loop(0, NVS, _vreg, carry)
        cnt, fc = lax.fori_loop(0, SEGS, _seg,
                                (jnp.int32(0), jnp.int32(0)))

        @pl.when(fc >= 7)
        def _():
            o = (fc + 1) & 7
            _gather_desc(o).wait()
            pltpu.sync_copy(rows_v.at[pl.ds(o * FB, FB)],
                            shared.at[vidx.at[o]], add=True)

        @pl.when(fc >= 6)
        def _():
            o = (fc + 2) & 7
            _gather_desc(o).wait()
            pltpu.sync_copy(rows_v.at[pl.ds(o * FB, FB)],
                            shared.at[vidx.at[o]], add=True)

        @pl.when(fc >= 5)
        def _():
            o = (fc + 3) & 7
            _gather_desc(o).wait()
            pltpu.sync_copy(rows_v.at[pl.ds(o * FB, FB)],
                            shared.at[vidx.at[o]], add=True)

        @pl.when(fc >= 4)
        def _():
            o = (fc + 4) & 7
            _gather_desc(o).wait()
            pltpu.sync_copy(rows_v.at[pl.ds(o * FB, FB)],
                            shared.at[vidx.at[o]], add=True)

        @pl.when(fc >= 3)
        def _():
            o = (fc + 5) & 7
            _gather_desc(o).wait()
            pltpu.sync_copy(rows_v.at[pl.ds(o * FB, FB)],
                            shared.at[vidx.at[o]], add=True)

        @pl.when(fc >= 2)
        def _():
            o = (fc + 6) & 7
            _gather_desc(o).wait()
            pltpu.sync_copy(rows_v.at[pl.ds(o * FB, FB)],
                            shared.at[vidx.at[o]], add=True)

        @pl.when(fc >= 1)
        def _():
            o = (fc + 7) & 7
            _gather_desc(o).wait()
            pltpu.sync_copy(rows_v.at[pl.ds(o * FB, FB)],
                            shared.at[vidx.at[o]], add=True)

        @pl.when(fc >= 8)
        def _():
            _scatter_desc(fc & 7).wait()

        for tt in range(FB // 16 + 1):
            vbuf[pl.ds(cnt + tt * 16, 16)] = zeros16i
            gbuf[pl.ds(cnt + tt * 16, 16)] = zrow16

        @pl.when(cnt > 0)
        def _():
            for tt in range(FB // 16):
                vidx[0, pl.ds(tt * 16, 16)] = vbuf[pl.ds(tt * 16, 16)]
                gidx[0, pl.ds(tt * 16, 16)] = gbuf[pl.ds(tt * 16, 16)]
            _gather_desc(0).start()
            _gather_desc(0).wait()
            pltpu.sync_copy(rows_v.at[pl.ds(0, FB)],
                            shared.at[vidx.at[0]], add=True)
        plsc.subcore_barrier()

        ob = lo + ob_local
        pltpu.sync_copy(shared.at[pl.ds(ob_local, 528)],
                        s_out.at[pl.ds(ob, 528)])
        plsc.subcore_barrier()


def _mm_body(x_ref, w_ref, t_ref, xc_ref):
    y = jnp.dot(x_ref[...], w_ref[...], preferred_element_type=jnp.float32)
    t_ref[...] = y[:, :D]
    for r in range(R):
        xc_ref[r] = y[:, D * (r + 1):D * (r + 2)]


_mm_call = pl.pallas_call(
    _mm_body,
    grid=(NP // BR,),
    in_specs=[
        pl.BlockSpec((BR, D), lambda i: (i, 0)),
        pl.BlockSpec((D, 7 * D), lambda i: (0, 0)),
    ],
    out_specs=[
        pl.BlockSpec((BR, D), lambda i: (i, 0)),
        pl.BlockSpec((R, BR, D), lambda i: (0, i, 0)),
    ],
    out_shape=[
        jax.ShapeDtypeStruct((NP, D), jnp.float32),
        jax.ShapeDtypeStruct((R, NP, D), jnp.float32),
    ],
)


def _gn(x, w, b):
    mu = jnp.mean(x, axis=1, keepdims=True)
    xc = x - mu
    v = jnp.mean(xc * xc, axis=1, keepdims=True)
    return xc * lax.rsqrt(v + 1e-5) * w + b


def _post_body(t0_ref, s_ref, res_ref, w2_ref, g1w, g1b, g2w, g2b, out_ref):
    t = t0_ref[...] + s_ref[...]
    h = jnp.maximum(_gn(t, g1w[...], g1b[...]), 0.0)
    y = jnp.dot(h, w2_ref[...], preferred_element_type=jnp.float32)
    o = _gn(y, g2w[...], g2b[...])
    out_ref[...] = jnp.maximum(o + res_ref[...], 0.0)


_vec_spec = pl.BlockSpec((1, D), lambda i: (0, 0))
_post_call = pl.pallas_call(
    _post_body,
    grid=(NP // BR,),
    in_specs=[
        pl.BlockSpec((BR, D), lambda i: (i, 0)),
        pl.BlockSpec((BR, D), lambda i: (i, 0)),
        pl.BlockSpec((BR, D), lambda i: (i, 0)),
        pl.BlockSpec((D, D), lambda i: (0, 0)),
        _vec_spec, _vec_spec, _vec_spec, _vec_spec,
    ],
    out_specs=pl.BlockSpec((BR, D), lambda i: (i, 0)),
    out_shape=jax.ShapeDtypeStruct((NP, D), jnp.float32),
)


def _postmm_body(t0_ref, s_ref, res_ref, w2_ref, g1w, g1b, g2w, g2b,
                 wcat_ref, t0n_ref, xcn_ref):
    t = t0_ref[...] + s_ref[...]
    h = jnp.maximum(_gn(t, g1w[...], g1b[...]), 0.0)
    y = jnp.dot(h, w2_ref[...], preferred_element_type=jnp.float32)
    o = _gn(y, g2w[...], g2b[...])
    f = jnp.maximum(o + res_ref[...], 0.0)
    y2 = jnp.dot(f, wcat_ref[...], preferred_element_type=jnp.float32)
    t0n_ref[...] = y2[:, :D]
    for r in range(R):
        xcn_ref[r] = y2[:, D * (r + 1):D * (r + 2)]


_postmm_call = pl.pallas_call(
    _postmm_body,
    grid=(NP // BR,),
    in_specs=[
        pl.BlockSpec((BR, D), lambda i: (i, 0)),
        pl.BlockSpec((BR, D), lambda i: (i, 0)),
        pl.BlockSpec((BR, D), lambda i: (i, 0)),
        pl.BlockSpec((D, D), lambda i: (0, 0)),
        _vec_spec, _vec_spec, _vec_spec, _vec_spec,
        pl.BlockSpec((D, 7 * D), lambda i: (0, 0)),
    ],
    out_specs=[
        pl.BlockSpec((BR, D), lambda i: (i, 0)),
        pl.BlockSpec((R, BR, D), lambda i: (0, i, 0)),
    ],
    out_shape=[
        jax.ShapeDtypeStruct((NP, D), jnp.float32),
        jax.ShapeDtypeStruct((R, NP, D), jnp.float32),
    ],
)


def kernel(feat,
           pre0_u, pre0_v, pre1_u, pre1_v, suc0_u, suc0_v, suc1_u, suc1_v,
           left_u, left_v, right_u, right_v,
           W_ctr_0, W_pre0_0, W_pre1_0, W_suc0_0, W_suc1_0, W_left_0,
           W_right_0, W_ctr2_0, gn1_w_0, gn1_b_0, gn2_w_0, gn2_b_0,
           W_ctr_1, W_pre0_1, W_pre1_1, W_suc0_1, W_suc1_1, W_left_1,
           W_right_1, W_ctr2_1, gn1_w_1, gn1_b_1, gn2_w_1, gn2_b_1):
    f32 = jnp.float32
    feat_p = jnp.zeros((NP, D), f32).at[:N].set(feat)
    res = feat_p

    us = [pre0_u, suc0_u, pre1_u, suc1_u, left_u, right_u]
    vs = [pre0_v, suc0_v, pre1_v, suc1_v, left_v, right_v]
    pad = ETP - E_TOT
    u_all = jnp.concatenate(
        [u.astype(jnp.int32) for u in us]
        + [jnp.full((pad,), PAD_U, jnp.int32)])
    g_all = jnp.concatenate(
        [vs[r].astype(jnp.int32) + r * NP for r in range(R)]
        + [jnp.zeros((pad,), jnp.int32)])

    blocks = [
        ([W_ctr_0, W_pre0_0, W_suc0_0, W_pre1_0, W_suc1_0, W_left_0,
          W_right_0], W_ctr2_0, gn1_w_0, gn1_b_0, gn2_w_0, gn2_b_0),
        ([W_ctr_1, W_pre0_1, W_suc0_1, W_pre1_1, W_suc1_1, W_left_1,
          W_right_1], W_ctr2_1, gn1_w_1, gn1_b_1, gn2_w_1, gn2_b_1),
    ]

    ws0, w2_0, g1w0, g1b0, g2w0, g2b0 = blocks[0]
    ws1, w2_1, g1w1, g1b1, g2w1, g2b1 = blocks[1]
    wcat0 = jnp.concatenate([w.T for w in ws0], axis=1)
    wcat1 = jnp.concatenate([w.T for w in ws1], axis=1)

    t0, xc = _mm_call(feat_p, wcat0)
    s0 = _sc_scatter(xc.reshape(R * NP, D), u_all, g_all)
    t0b, xcb = _postmm_call(t0, s0, res, w2_0.T,
                            g1w0.reshape(1, D), g1b0.reshape(1, D),
                            g2w0.reshape(1, D), g2b0.reshape(1, D), wcat1)
    s1 = _sc_scatter(xcb.reshape(R * NP, D), u_all, g_all)
    f = _post_call(t0b, s1, res, w2_1.T,
                   g1w1.reshape(1, D), g1b1.reshape(1, D),
                   g2w1.reshape(1, D), g2b1.reshape(1, D))
    return f[:N]


# speedup vs baseline: 1.5596x; 1.5596x over previous
import functools

import jax
import jax.numpy as jnp
from jax import lax
from jax.experimental import pallas as pl
from jax.experimental.pallas import tpu as pltpu
from jax.experimental.pallas import tpu_sc as plsc

N = 50000
D = 128
R = 6
NP = 50176
CH = 12544
SH = CH
ZROW = 50000
E_TOT = 300000
EPT = 18944
ETP = 16 * EPT
SEG = 592
SEGS = EPT // SEG
NVS = SEG // 16
FB = 32
BR = 1792
PAD_U = 1 << 20

_mesh = plsc.VectorSubcoreMesh(
    core_axis_name="c", subcore_axis_name="s", num_cores=2, num_subcores=16
)


@functools.partial(
    pl.kernel,
    out_type=jax.ShapeDtypeStruct((NP, D), jnp.float32),
    mesh=_mesh,
    compiler_params=pltpu.CompilerParams(needs_layout_passes=False),
    scratch_types=[
        pltpu.VMEM((2 * SEG,), jnp.int32),
        pltpu.VMEM((2 * SEG,), jnp.int32),
        pltpu.VMEM((96,), jnp.int32),
        pltpu.VMEM((96,), jnp.int32),
        pltpu.VMEM((6, FB), jnp.int32),
        pltpu.VMEM((6, FB), jnp.int32),
        pltpu.VMEM((6 * FB, D), jnp.float32),
        pltpu.VMEM_SHARED((SH, D), jnp.float32),
        pltpu.SemaphoreType.DMA,
        pltpu.SemaphoreType.DMA,
        pltpu.SemaphoreType.DMA,
    ],
)
def _sc_scatter(xcat, u_all, g_all, s_out,
                u_seg, g_seg, vbuf, gbuf, vidx, gidx, rows_v, shared,
                sem, sem2, sem3):
    c = lax.axis_index("c")
    s = lax.axis_index("s")
    ones16 = jnp.ones((16,), jnp.int32)
    zeros16i = jnp.zeros((16,), jnp.int32)
    zrow16 = jnp.full((16,), ZROW, jnp.int32)

    ebase = s * EPT
    zb = s * 784
    ob_local = s * 784

    def _gather_desc(slot):
        return pltpu.make_async_copy(
            xcat.at[gidx.at[slot]], rows_v.at[pl.ds(slot * FB, FB)], sem)

    def _scatter_desc(slot):
        return pltpu.make_async_copy(
            rows_v.at[pl.ds(slot * FB, FB)], shared.at[vidx.at[slot]], sem2)

    for lc in range(2):
        chunk = 2 * c + lc
        lo = chunk * CH

        zeros16f = jnp.zeros((16,), jnp.float32)
        for i in range(16):
            for j in range(8):
                rows_v[i, pl.ds(j * 16, 16)] = zeros16f

        def _zero(k, _):
            pltpu.sync_copy(rows_v.at[pl.ds(0, 16)],
                            shared.at[pl.ds(zb + k * 16, 16)])
            return 0
        lax.fori_loop(0, 49, _zero, 0)
        plsc.subcore_barrier()

        def _eload(si, eslot):
            pltpu.async_copy(u_all.at[pl.ds(ebase + si * SEG, SEG)],
                             u_seg.at[pl.ds(eslot * SEG, SEG)], sem3)
            pltpu.async_copy(g_all.at[pl.ds(ebase + si * SEG, SEG)],
                             g_seg.at[pl.ds(eslot * SEG, SEG)], sem3)

        def _ewait(si, eslot):
            pltpu.make_async_copy(u_all.at[pl.ds(ebase + si * SEG, SEG)],
                                  u_seg.at[pl.ds(eslot * SEG, SEG)],
                                  sem3).wait()
            pltpu.make_async_copy(g_all.at[pl.ds(ebase + si * SEG, SEG)],
                                  g_seg.at[pl.ds(eslot * SEG, SEG)],
                                  sem3).wait()

        _eload(0, 0)

        def _seg(si, carry):
            eslot = si & 1
            _ewait(si, eslot)

            @pl.when(si < SEGS - 1)
            def _():
                _eload(si + 1, 1 - eslot)

            def _vreg(i, carry):
                cnt, fc = carry
                u16 = u_seg[pl.ds(eslot * SEG + i * 16, 16)]
                g16 = g_seg[pl.ds(eslot * SEG + i * 16, 16)]
                m = (u16 >= lo) & (u16 < lo + CH)
                m32 = jnp.where(m, ones16, zeros16i)
                pos = cnt + plsc.cumsum(m32) - 1
                plsc.store_scatter(vbuf, [pos], u16 - lo, mask=m)
                plsc.store_scatter(gbuf, [pos], g16, mask=m)
                cnt2 = cnt + jnp.sum(m32)

                @pl.when(cnt2 >= FB)
                def _():
                    slot = lax.rem(fc, 6)
                    prev5 = lax.rem(fc + 1, 6)

                    @pl.when(fc >= 6)
                    def _():
                        _scatter_desc(slot).wait()
                    for tt in range(FB // 16):
                        vidx[slot, pl.ds(tt * 16, 16)] = \
                            vbuf[pl.ds(tt * 16, 16)]
                        gidx[slot, pl.ds(tt * 16, 16)] = \
                            gbuf[pl.ds(tt * 16, 16)]
                    vbuf[pl.ds(0, 16)] = vbuf[pl.ds(FB, 16)]
                    gbuf[pl.ds(0, 16)] = gbuf[pl.ds(FB, 16)]

                    @pl.when(fc >= 5)
                    def _():
                        _gather_desc(prev5).wait()
                        pltpu.async_copy(
                            rows_v.at[pl.ds(prev5 * FB, FB)],
                            shared.at[vidx.at[prev5]], sem2, add=True)
                    pltpu.async_copy(
                        xcat.at[gidx.at[slot]],
                        rows_v.at[pl.ds(slot * FB, FB)], sem)
                hit = cnt2 >= FB
                return (jnp.where(hit, cnt2 - FB, cnt2),
                        jnp.where(hit, fc + 1, fc))
            return lax.fori_loop(0, NVS, _vreg, carry)
        cnt, fc = lax.fori_loop(0, SEGS, _seg,
                                (jnp.int32(0), jnp.int32(0)))

        @pl.when(fc >= 5)
        def _():
            o = lax.rem(fc + 1, 6)
            _gather_desc(o).wait()
            pltpu.sync_copy(rows_v.at[pl.ds(o * FB, FB)],
                            shared.at[vidx.at[o]], add=True)

        @pl.when(fc >= 4)
        def _():
            o = lax.rem(fc + 2, 6)
            _gather_desc(o).wait()
            pltpu.sync_copy(rows_v.at[pl.ds(o * FB, FB)],
                            shared.at[vidx.at[o]], add=True)

        @pl.when(fc >= 3)
        def _():
            o = lax.rem(fc + 3, 6)
            _gather_desc(o).wait()
            pltpu.sync_copy(rows_v.at[pl.ds(o * FB, FB)],
                            shared.at[vidx.at[o]], add=True)

        @pl.when(fc >= 2)
        def _():
            o = lax.rem(fc + 4, 6)
            _gather_desc(o).wait()
            pltpu.sync_copy(rows_v.at[pl.ds(o * FB, FB)],
                            shared.at[vidx.at[o]], add=True)

        @pl.when(fc >= 1)
        def _():
            o = lax.rem(fc + 5, 6)
            _gather_desc(o).wait()
            pltpu.sync_copy(rows_v.at[pl.ds(o * FB, FB)],
                            shared.at[vidx.at[o]], add=True)

        @pl.when(fc >= 6)
        def _():
            _scatter_desc(lax.rem(fc, 6)).wait()

        for tt in range(FB // 16 + 1):
            vbuf[pl.ds(cnt + tt * 16, 16)] = zeros16i
            gbuf[pl.ds(cnt + tt * 16, 16)] = zrow16

        @pl.when(cnt > 0)
        def _():
            for tt in range(FB // 16):
                vidx[0, pl.ds(tt * 16, 16)] = vbuf[pl.ds(tt * 16, 16)]
                gidx[0, pl.ds(tt * 16, 16)] = gbuf[pl.ds(tt * 16, 16)]
            _gather_desc(0).start()
            _gather_desc(0).wait()
            pltpu.sync_copy(rows_v.at[pl.ds(0, FB)],
                            shared.at[vidx.at[0]], add=True)
        plsc.subcore_barrier()

        ob = lo + ob_local
        pltpu.sync_copy(shared.at[pl.ds(ob_local, 784)],
                        s_out.at[pl.ds(ob, 784)])
        plsc.subcore_barrier()


def _mm_body(x_ref, w_ref, t_ref, xc_ref):
    y = jnp.dot(x_ref[...], w_ref[...], preferred_element_type=jnp.float32)
    t_ref[...] = y[:, :D]
    for r in range(R):
        xc_ref[r] = y[:, D * (r + 1):D * (r + 2)]


_mm_call = pl.pallas_call(
    _mm_body,
    grid=(NP // BR,),
    in_specs=[
        pl.BlockSpec((BR, D), lambda i: (i, 0)),
        pl.BlockSpec((D, 7 * D), lambda i: (0, 0)),
    ],
    out_specs=[
        pl.BlockSpec((BR, D), lambda i: (i, 0)),
        pl.BlockSpec((R, BR, D), lambda i: (0, i, 0)),
    ],
    out_shape=[
        jax.ShapeDtypeStruct((NP, D), jnp.float32),
        jax.ShapeDtypeStruct((R, NP, D), jnp.float32),
    ],
)


def _gn(x, w, b):
    mu = jnp.mean(x, axis=1, keepdims=True)
    xc = x - mu
    v = jnp.mean(xc * xc, axis=1, keepdims=True)
    return xc * lax.rsqrt(v + 1e-5) * w + b


def _post_body(t0_ref, s_ref, res_ref, w2_ref, g1w, g1b, g2w, g2b, out_ref):
    t = t0_ref[...] + s_ref[...]
    h = jnp.maximum(_gn(t, g1w[...], g1b[...]), 0.0)
    y = jnp.dot(h, w2_ref[...], preferred_element_type=jnp.float32)
    o = _gn(y, g2w[...], g2b[...])
    out_ref[...] = jnp.maximum(o + res_ref[...], 0.0)


_vec_spec = pl.BlockSpec((1, D), lambda i: (0, 0))
_post_call = pl.pallas_call(
    _post_body,
    grid=(NP // BR,),
    in_specs=[
        pl.BlockSpec((BR, D), lambda i: (i, 0)),
        pl.BlockSpec((BR, D), lambda i: (i, 0)),
        pl.BlockSpec((BR, D), lambda i: (i, 0)),
        pl.BlockSpec((D, D), lambda i: (0, 0)),
        _vec_spec, _vec_spec, _vec_spec, _vec_spec,
    ],
    out_specs=pl.BlockSpec((BR, D), lambda i: (i, 0)),
    out_shape=jax.ShapeDtypeStruct((NP, D), jnp.float32),
)


def _postmm_body(t0_ref, s_ref, res_ref, w2_ref, g1w, g1b, g2w, g2b,
                 wcat_ref, t0n_ref, xcn_ref):
    t = t0_ref[...] + s_ref[...]
    h = jnp.maximum(_gn(t, g1w[...], g1b[...]), 0.0)
    y = jnp.dot(h, w2_ref[...], preferred_element_type=jnp.float32)
    o = _gn(y, g2w[...], g2b[...])
    f = jnp.maximum(o + res_ref[...], 0.0)
    y2 = jnp.dot(f, wcat_ref[...], preferred_element_type=jnp.float32)
    t0n_ref[...] = y2[:, :D]
    for r in range(R):
        xcn_ref[r] = y2[:, D * (r + 1):D * (r + 2)]


_postmm_call = pl.pallas_call(
    _postmm_body,
    grid=(NP // BR,),
    in_specs=[
        pl.BlockSpec((BR, D), lambda i: (i, 0)),
        pl.BlockSpec((BR, D), lambda i: (i, 0)),
        pl.BlockSpec((BR, D), lambda i: (i, 0)),
        pl.BlockSpec((D, D), lambda i: (0, 0)),
        _vec_spec, _vec_spec, _vec_spec, _vec_spec,
        pl.BlockSpec((D, 7 * D), lambda i: (0, 0)),
    ],
    out_specs=[
        pl.BlockSpec((BR, D), lambda i: (i, 0)),
        pl.BlockSpec((R, BR, D), lambda i: (0, i, 0)),
    ],
    out_shape=[
        jax.ShapeDtypeStruct((NP, D), jnp.float32),
        jax.ShapeDtypeStruct((R, NP, D), jnp.float32),
    ],
)


def kernel(feat,
           pre0_u, pre0_v, pre1_u, pre1_v, suc0_u, suc0_v, suc1_u, suc1_v,
           left_u, left_v, right_u, right_v,
           W_ctr_0, W_pre0_0, W_pre1_0, W_suc0_0, W_suc1_0, W_left_0,
           W_right_0, W_ctr2_0, gn1_w_0, gn1_b_0, gn2_w_0, gn2_b_0,
           W_ctr_1, W_pre0_1, W_pre1_1, W_suc0_1, W_suc1_1, W_left_1,
           W_right_1, W_ctr2_1, gn1_w_1, gn1_b_1, gn2_w_1, gn2_b_1):
    f32 = jnp.float32
    feat_p = jnp.zeros((NP, D), f32).at[:N].set(feat)
    res = feat_p

    us = [pre0_u, suc0_u, pre1_u, suc1_u, left_u, right_u]
    vs = [pre0_v, suc0_v, pre1_v, suc1_v, left_v, right_v]
    pad = ETP - E_TOT
    u_all = jnp.concatenate(
        [u.astype(jnp.int32) for u in us]
        + [jnp.full((pad,), PAD_U, jnp.int32)])
    g_all = jnp.concatenate(
        [vs[r].astype(jnp.int32) + r * NP for r in range(R)]
        + [jnp.zeros((pad,), jnp.int32)])

    blocks = [
        ([W_ctr_0, W_pre0_0, W_suc0_0, W_pre1_0, W_suc1_0, W_left_0,
          W_right_0], W_ctr2_0, gn1_w_0, gn1_b_0, gn2_w_0, gn2_b_0),
        ([W_ctr_1, W_pre0_1, W_suc0_1, W_pre1_1, W_suc1_1, W_left_1,
          W_right_1], W_ctr2_1, gn1_w_1, gn1_b_1, gn2_w_1, gn2_b_1),
    ]

    ws0, w2_0, g1w0, g1b0, g2w0, g2b0 = blocks[0]
    ws1, w2_1, g1w1, g1b1, g2w1, g2b1 = blocks[1]
    wcat0 = jnp.concatenate([w.T for w in ws0], axis=1)
    wcat1 = jnp.concatenate([w.T for w in ws1], axis=1)

    t0, xc = _mm_call(feat_p, wcat0)
    s0 = _sc_scatter(xc.reshape(R * NP, D), u_all, g_all)
    t0b, xcb = _postmm_call(t0, s0, res, w2_0.T,
                            g1w0.reshape(1, D), g1b0.reshape(1, D),
                            g2w0.reshape(1, D), g2b0.reshape(1, D), wcat1)
    s1 = _sc_scatter(xcb.reshape(R * NP, D), u_all, g_all)
    f = _post_call(t0b, s1, res, w2_1.T,
                   g1w1.reshape(1, D), g1b1.reshape(1, D),
                   g2w1.reshape(1, D), g2b1.reshape(1, D))
    return f[:N]


# speedup vs baseline: 1.5834x; 1.0152x over previous
import functools

import jax
import jax.numpy as jnp
from jax import lax
from jax.experimental import pallas as pl
from jax.experimental.pallas import tpu as pltpu
from jax.experimental.pallas import tpu_sc as plsc

N = 50000
D = 128
R = 6
NP = 50176
CH = 12544
SH = CH
ZROW = 50000
E_TOT = 300000
EPT = 18944
ETP = 16 * EPT
SEG = 592
SEGS = EPT // SEG
NVS = SEG // 16
FB = 32
BR = 1792
PAD_U = 1 << 20

_mesh = plsc.VectorSubcoreMesh(
    core_axis_name="c", subcore_axis_name="s", num_cores=2, num_subcores=16
)


@functools.partial(
    pl.kernel,
    out_type=jax.ShapeDtypeStruct((NP, D), jnp.float32),
    mesh=_mesh,
    compiler_params=pltpu.CompilerParams(needs_layout_passes=False),
    scratch_types=[
        pltpu.VMEM((2 * SEG,), jnp.int32),
        pltpu.VMEM((2 * SEG,), jnp.int32),
        pltpu.VMEM((96,), jnp.int32),
        pltpu.VMEM((96,), jnp.int32),
        pltpu.VMEM((6, FB), jnp.int32),
        pltpu.VMEM((6, FB), jnp.int32),
        pltpu.VMEM((6 * FB, D), jnp.float32),
        pltpu.VMEM_SHARED((SH, D), jnp.float32),
        pltpu.SemaphoreType.DMA,
        pltpu.SemaphoreType.DMA,
        pltpu.SemaphoreType.DMA,
    ],
)
def _sc_scatter(xcat, u_all, g_all, s_out,
                u_seg, g_seg, vbuf, gbuf, vidx, gidx, rows_v, shared,
                sem, sem2, sem3):
    c = lax.axis_index("c")
    s = lax.axis_index("s")
    ones16 = jnp.ones((16,), jnp.int32)
    zeros16i = jnp.zeros((16,), jnp.int32)
    zrow16 = jnp.full((16,), ZROW, jnp.int32)

    ebase = s * EPT
    zb = s * 784
    ob_local = s * 784

    def _gather_desc(slot):
        return pltpu.make_async_copy(
            xcat.at[gidx.at[slot]], rows_v.at[pl.ds(slot * FB, FB)], sem)

    def _scatter_desc(slot):
        return pltpu.make_async_copy(
            rows_v.at[pl.ds(slot * FB, FB)], shared.at[vidx.at[slot]], sem2)

    for lc in range(2):
        chunk = 2 * c + lc
        lo = chunk * CH

        zeros16f = jnp.zeros((16,), jnp.float32)
        for i in range(16):
            for j in range(8):
                rows_v[i, pl.ds(j * 16, 16)] = zeros16f

        def _zero(k, _):
            pltpu.async_copy(rows_v.at[pl.ds(0, 16)],
                             shared.at[pl.ds(zb + k * 16, 16)], sem3)
            return 0
        lax.fori_loop(0, 49, _zero, 0)

        def _zwait(k, _):
            pltpu.make_async_copy(rows_v.at[pl.ds(0, 16)],
                                  shared.at[pl.ds(zb + k * 16, 16)],
                                  sem3).wait()
            return 0
        lax.fori_loop(0, 49, _zwait, 0)
        plsc.subcore_barrier()

        def _eload(si, eslot):
            pltpu.async_copy(u_all.at[pl.ds(ebase + si * SEG, SEG)],
                             u_seg.at[pl.ds(eslot * SEG, SEG)], sem3)
            pltpu.async_copy(g_all.at[pl.ds(ebase + si * SEG, SEG)],
                             g_seg.at[pl.ds(eslot * SEG, SEG)], sem3)

        def _ewait(si, eslot):
            pltpu.make_async_copy(u_all.at[pl.ds(ebase + si * SEG, SEG)],
                                  u_seg.at[pl.ds(eslot * SEG, SEG)],
                                  sem3).wait()
            pltpu.make_async_copy(g_all.at[pl.ds(ebase + si * SEG, SEG)],
                                  g_seg.at[pl.ds(eslot * SEG, SEG)],
                                  sem3).wait()

        _eload(0, 0)

        def _seg(si, carry):
            eslot = si & 1
            _ewait(si, eslot)

            @pl.when(si < SEGS - 1)
            def _():
                _eload(si + 1, 1 - eslot)

            def _vreg(i, carry):
                cnt, fc = carry
                u16 = u_seg[pl.ds(eslot * SEG + i * 16, 16)]
                g16 = g_seg[pl.ds(eslot * SEG + i * 16, 16)]
                m = (u16 >= lo) & (u16 < lo + CH)
                m32 = jnp.where(m, ones16, zeros16i)
                pos = cnt + plsc.cumsum(m32) - 1
                plsc.store_scatter(vbuf, [pos], u16 - lo, mask=m)
                plsc.store_scatter(gbuf, [pos], g16, mask=m)
                cnt2 = cnt + jnp.sum(m32)

                @pl.when(cnt2 >= FB)
                def _():
                    slot = lax.rem(fc, 6)
                    prev5 = lax.rem(fc + 1, 6)

                    @pl.when(fc >= 6)
                    def _():
                        _scatter_desc(slot).wait()
                    for tt in range(FB // 16):
                        vidx[slot, pl.ds(tt * 16, 16)] = \
                            vbuf[pl.ds(tt * 16, 16)]
                        gidx[slot, pl.ds(tt * 16, 16)] = \
                            gbuf[pl.ds(tt * 16, 16)]
                    vbuf[pl.ds(0, 16)] = vbuf[pl.ds(FB, 16)]
                    gbuf[pl.ds(0, 16)] = gbuf[pl.ds(FB, 16)]

                    @pl.when(fc >= 5)
                    def _():
                        _gather_desc(prev5).wait()
                        pltpu.async_copy(
                            rows_v.at[pl.ds(prev5 * FB, FB)],
                            shared.at[vidx.at[prev5]], sem2, add=True)
                    pltpu.async_copy(
                        xcat.at[gidx.at[slot]],
                        rows_v.at[pl.ds(slot * FB, FB)], sem)
                hit = cnt2 >= FB
                return (jnp.where(hit, cnt2 - FB, cnt2),
                        jnp.where(hit, fc + 1, fc))
            return lax.fori_loop(0, NVS, _vreg, carry)
        cnt, fc = lax.fori_loop(0, SEGS, _seg,
                                (jnp.int32(0), jnp.int32(0)))

        @pl.when(fc >= 5)
        def _():
            o = lax.rem(fc + 1, 6)
            _gather_desc(o).wait()
            pltpu.sync_copy(rows_v.at[pl.ds(o * FB, FB)],
                            shared.at[vidx.at[o]], add=True)

        @pl.when(fc >= 4)
        def _():
            o = lax.rem(fc + 2, 6)
            _gather_desc(o).wait()
            pltpu.sync_copy(rows_v.at[pl.ds(o * FB, FB)],
                            shared.at[vidx.at[o]], add=True)

        @pl.when(fc >= 3)
        def _():
            o = lax.rem(fc + 3, 6)
            _gather_desc(o).wait()
            pltpu.sync_copy(rows_v.at[pl.ds(o * FB, FB)],
                            shared.at[vidx.at[o]], add=True)

        @pl.when(fc >= 2)
        def _():
            o = lax.rem(fc + 4, 6)
            _gather_desc(o).wait()
            pltpu.sync_copy(rows_v.at[pl.ds(o * FB, FB)],
                            shared.at[vidx.at[o]], add=True)

        @pl.when(fc >= 1)
        def _():
            o = lax.rem(fc + 5, 6)
            _gather_desc(o).wait()
            pltpu.sync_copy(rows_v.at[pl.ds(o * FB, FB)],
                            shared.at[vidx.at[o]], add=True)

        @pl.when(fc >= 6)
        def _():
            _scatter_desc(lax.rem(fc, 6)).wait()

        for tt in range(FB // 16 + 1):
            vbuf[pl.ds(cnt + tt * 16, 16)] = zeros16i
            gbuf[pl.ds(cnt + tt * 16, 16)] = zrow16

        @pl.when(cnt > 0)
        def _():
            for tt in range(FB // 16):
                vidx[0, pl.ds(tt * 16, 16)] = vbuf[pl.ds(tt * 16, 16)]
                gidx[0, pl.ds(tt * 16, 16)] = gbuf[pl.ds(tt * 16, 16)]
            _gather_desc(0).start()
            _gather_desc(0).wait()
            pltpu.sync_copy(rows_v.at[pl.ds(0, FB)],
                            shared.at[vidx.at[0]], add=True)
        plsc.subcore_barrier()

        ob = lo + ob_local
        pltpu.sync_copy(shared.at[pl.ds(ob_local, 784)],
                        s_out.at[pl.ds(ob, 784)])
        plsc.subcore_barrier()


def _mm_body(x_ref, w_ref, t_ref, xc_ref):
    y = jnp.dot(x_ref[...], w_ref[...], preferred_element_type=jnp.float32)
    t_ref[...] = y[:, :D]
    for r in range(R):
        xc_ref[r] = y[:, D * (r + 1):D * (r + 2)]


_mm_call = pl.pallas_call(
    _mm_body,
    grid=(NP // BR,),
    in_specs=[
        pl.BlockSpec((BR, D), lambda i: (i, 0)),
        pl.BlockSpec((D, 7 * D), lambda i: (0, 0)),
    ],
    out_specs=[
        pl.BlockSpec((BR, D), lambda i: (i, 0)),
        pl.BlockSpec((R, BR, D), lambda i: (0, i, 0)),
    ],
    out_shape=[
        jax.ShapeDtypeStruct((NP, D), jnp.float32),
        jax.ShapeDtypeStruct((R, NP, D), jnp.float32),
    ],
)


def _gn(x, w, b):
    mu = jnp.mean(x, axis=1, keepdims=True)
    xc = x - mu
    v = jnp.mean(xc * xc, axis=1, keepdims=True)
    return xc * lax.rsqrt(v + 1e-5) * w + b


def _post_body(t0_ref, s_ref, res_ref, w2_ref, g1w, g1b, g2w, g2b, out_ref):
    t = t0_ref[...] + s_ref[...]
    h = jnp.maximum(_gn(t, g1w[...], g1b[...]), 0.0)
    y = jnp.dot(h, w2_ref[...], preferred_element_type=jnp.float32)
    o = _gn(y, g2w[...], g2b[...])
    out_ref[...] = jnp.maximum(o + res_ref[...], 0.0)


_vec_spec = pl.BlockSpec((1, D), lambda i: (0, 0))
_post_call = pl.pallas_call(
    _post_body,
    grid=(NP // BR,),
    in_specs=[
        pl.BlockSpec((BR, D), lambda i: (i, 0)),
        pl.BlockSpec((BR, D), lambda i: (i, 0)),
        pl.BlockSpec((BR, D), lambda i: (i, 0)),
        pl.BlockSpec((D, D), lambda i: (0, 0)),
        _vec_spec, _vec_spec, _vec_spec, _vec_spec,
    ],
    out_specs=pl.BlockSpec((BR, D), lambda i: (i, 0)),
    out_shape=jax.ShapeDtypeStruct((NP, D), jnp.float32),
)


def _postmm_body(t0_ref, s_ref, res_ref, w2_ref, g1w, g1b, g2w, g2b,
                 wcat_ref, t0n_ref, xcn_ref):
    t = t0_ref[...] + s_ref[...]
    h = jnp.maximum(_gn(t, g1w[...], g1b[...]), 0.0)
    y = jnp.dot(h, w2_ref[...], preferred_element_type=jnp.float32)
    o = _gn(y, g2w[...], g2b[...])
    f = jnp.maximum(o + res_ref[...], 0.0)
    y2 = jnp.dot(f, wcat_ref[...], preferred_element_type=jnp.float32)
    t0n_ref[...] = y2[:, :D]
    for r in range(R):
        xcn_ref[r] = y2[:, D * (r + 1):D * (r + 2)]


_postmm_call = pl.pallas_call(
    _postmm_body,
    grid=(NP // BR,),
    in_specs=[
        pl.BlockSpec((BR, D), lambda i: (i, 0)),
        pl.BlockSpec((BR, D), lambda i: (i, 0)),
        pl.BlockSpec((BR, D), lambda i: (i, 0)),
        pl.BlockSpec((D, D), lambda i: (0, 0)),
        _vec_spec, _vec_spec, _vec_spec, _vec_spec,
        pl.BlockSpec((D, 7 * D), lambda i: (0, 0)),
    ],
    out_specs=[
        pl.BlockSpec((BR, D), lambda i: (i, 0)),
        pl.BlockSpec((R, BR, D), lambda i: (0, i, 0)),
    ],
    out_shape=[
        jax.ShapeDtypeStruct((NP, D), jnp.float32),
        jax.ShapeDtypeStruct((R, NP, D), jnp.float32),
    ],
)


def kernel(feat,
           pre0_u, pre0_v, pre1_u, pre1_v, suc0_u, suc0_v, suc1_u, suc1_v,
           left_u, left_v, right_u, right_v,
           W_ctr_0, W_pre0_0, W_pre1_0, W_suc0_0, W_suc1_0, W_left_0,
           W_right_0, W_ctr2_0, gn1_w_0, gn1_b_0, gn2_w_0, gn2_b_0,
           W_ctr_1, W_pre0_1, W_pre1_1, W_suc0_1, W_suc1_1, W_left_1,
           W_right_1, W_ctr2_1, gn1_w_1, gn1_b_1, gn2_w_1, gn2_b_1):
    f32 = jnp.float32
    feat_p = jnp.zeros((NP, D), f32).at[:N].set(feat)
    res = feat_p

    us = [pre0_u, suc0_u, pre1_u, suc1_u, left_u, right_u]
    vs = [pre0_v, suc0_v, pre1_v, suc1_v, left_v, right_v]
    pad = ETP - E_TOT
    u_all = jnp.concatenate(
        [u.astype(jnp.int32) for u in us]
        + [jnp.full((pad,), PAD_U, jnp.int32)])
    g_all = jnp.concatenate(
        [vs[r].astype(jnp.int32) + r * NP for r in range(R)]
        + [jnp.zeros((pad,), jnp.int32)])

    blocks = [
        ([W_ctr_0, W_pre0_0, W_suc0_0, W_pre1_0, W_suc1_0, W_left_0,
          W_right_0], W_ctr2_0, gn1_w_0, gn1_b_0, gn2_w_0, gn2_b_0),
        ([W_ctr_1, W_pre0_1, W_suc0_1, W_pre1_1, W_suc1_1, W_left_1,
          W_right_1], W_ctr2_1, gn1_w_1, gn1_b_1, gn2_w_1, gn2_b_1),
    ]

    ws0, w2_0, g1w0, g1b0, g2w0, g2b0 = blocks[0]
    ws1, w2_1, g1w1, g1b1, g2w1, g2b1 = blocks[1]
    wcat0 = jnp.concatenate([w.T for w in ws0], axis=1)
    wcat1 = jnp.concatenate([w.T for w in ws1], axis=1)

    t0, xc = _mm_call(feat_p, wcat0)
    s0 = _sc_scatter(xc.reshape(R * NP, D), u_all, g_all)
    t0b, xcb = _postmm_call(t0, s0, res, w2_0.T,
                            g1w0.reshape(1, D), g1b0.reshape(1, D),
                            g2w0.reshape(1, D), g2b0.reshape(1, D), wcat1)
    s1 = _sc_scatter(xcb.reshape(R * NP, D), u_all, g_all)
    f = _post_call(t0b, s1, res, w2_1.T,
                   g1w1.reshape(1, D), g1b1.reshape(1, D),
                   g2w1.reshape(1, D), g2b1.reshape(1, D))
    return f[:N]
